# baseline (speedup 1.0000x reference)
import functools

import jax
import jax.numpy as jnp
from jax import lax
from jax.experimental import pallas as pl
from jax.experimental.pallas import tpu as pltpu
from jax.experimental.pallas import tpu_sc as plsc

B, L, D = 4096, 200, 64
LD = L * D
NC, NS = 2, 16
NW = NC * NS
ROWS_PER_W = B // NW
NBUF = 4
NG = ROWS_PER_W // NBUF
UNROLL = 8
NVEC = LD // 16


def _sc_body(x_hbm, pe_hbm, o_hbm, xbuf, obuf, pebuf, insems, outsems):
    wid = lax.axis_index("s") * NC + lax.axis_index("c")
    base = wid * ROWS_PER_W

    pltpu.sync_copy(pe_hbm, pebuf)

    def in_copy(row, b):
        return pltpu.make_async_copy(
            x_hbm.at[pl.ds(row * LD, LD)], xbuf.at[b], insems.at[b]
        )

    def out_copy(row, b):
        return pltpu.make_async_copy(
            obuf.at[b], o_hbm.at[pl.ds(row * LD, LD)], outsems.at[b]
        )

    for b in range(NBUF):
        in_copy(base + b, b).start()

    def outer(g, carry):
        row0 = base + g * NBUF
        for b in range(NBUF):
            in_copy(row0 + b, b).wait()

            @pl.when(g > 0)
            def _():
                out_copy(row0 - NBUF + b, b).wait()

            xb = xbuf.at[b]
            ob = obuf.at[b]

            def add_chunk(j, carry2):
                for u in range(UNROLL):
                    off = (j * UNROLL + u) * 16
                    ob[pl.ds(off, 16)] = xb[pl.ds(off, 16)] + pebuf[pl.ds(off, 16)]
                return carry2

            lax.fori_loop(0, NVEC // UNROLL, add_chunk, 0)

            out_copy(row0 + b, b).start()

            @pl.when(g < NG - 1)
            def _():
                in_copy(row0 + NBUF + b, b).start()
        return carry

    lax.fori_loop(0, NG, outer, 0)

    for b in range(NBUF):
        out_copy(base + (NG - 1) * NBUF + b, b).wait()


def kernel(x, pos_emb):
    Bb, Ll, Dd = x.shape
    x_flat = x.reshape(Bb * Ll * Dd)
    pe_flat = pos_emb[:Ll].reshape(Ll * Dd)

    mesh = plsc.VectorSubcoreMesh(core_axis_name="c", subcore_axis_name="s")
    k = functools.partial(
        pl.kernel,
        mesh=mesh,
        out_type=jax.ShapeDtypeStruct((Bb * Ll * Dd,), jnp.float32),
        scratch_types=[
            pltpu.MemorySpace.VMEM((NBUF, LD), jnp.float32),
            pltpu.MemorySpace.VMEM((NBUF, LD), jnp.float32),
            pltpu.MemorySpace.VMEM((LD,), jnp.float32),
            pltpu.SemaphoreType.DMA((NBUF,)),
            pltpu.SemaphoreType.DMA((NBUF,)),
        ],
    )(_sc_body)
    out = k(x_flat, pe_flat)
    return out.reshape(Bb, Ll, Dd)
